# baseline (speedup 1.0000x reference)
import jax
import jax.numpy as jnp
from jax.experimental import pallas as pl
from jax.experimental.pallas import tpu as pltpu


def _pe_kernel(col_ref, row_ref, out_ref, scratch):
    half = col_ref.shape[1]
    H = row_ref.shape[0]
    W = col_ref.shape[0]

    @pl.when(pl.program_id(0) == 0)
    def _():
        col_t = col_ref[...].T
        row_t = row_ref[...].T
        scratch[:half] = jnp.broadcast_to(col_t[:, None, :], (half, H, W))
        scratch[half:] = jnp.broadcast_to(row_t[:, :, None], (half, H, W))

    out_ref[0] = scratch[...]


def kernel(image_feature, col_table, row_table):
    B, C, H, W = image_feature.shape
    half = col_table.shape[1]

    return pl.pallas_call(
        _pe_kernel,
        grid=(B,),
        in_specs=[
            pl.BlockSpec((W, half), lambda b: (0, 0)),
            pl.BlockSpec((H, half), lambda b: (0, 0)),
        ],
        out_specs=pl.BlockSpec((1, C, H, W), lambda b: (b, 0, 0, 0)),
        out_shape=jax.ShapeDtypeStruct((B, C, H, W), jnp.float32),
        scratch_shapes=[pltpu.VMEM((C, H, W), jnp.float32)],
    )(col_table, row_table)


# speedup vs baseline: 2.4070x; 2.4070x over previous
import jax
import jax.numpy as jnp
from jax.experimental import pallas as pl
from jax.experimental.pallas import tpu as pltpu


def _pe_kernel(col_ref, row_ref, out_ref, scratch):
    half = col_ref.shape[1]
    W = col_ref.shape[0]
    H = row_ref.shape[0]
    HW = H * W

    @pl.when(pl.program_id(0) == 0)
    def _():
        j = jax.lax.broadcasted_iota(jnp.int32, (W, HW), 1)
        i = jax.lax.broadcasted_iota(jnp.int32, (W, HW), 0)
        s_col = (jax.lax.rem(j, W) == i).astype(jnp.float32)
        s_row = (jax.lax.div(j, W) == i).astype(jnp.float32)
        col_t = col_ref[...].T
        row_t = row_ref[...].T
        scratch[:half] = jax.lax.dot(
            col_t, s_col, precision=jax.lax.Precision.HIGHEST,
            preferred_element_type=jnp.float32)
        scratch[half:] = jax.lax.dot(
            row_t, s_row, precision=jax.lax.Precision.HIGHEST,
            preferred_element_type=jnp.float32)

    out_ref[0] = scratch[...]


def kernel(image_feature, col_table, row_table):
    B, C, H, W = image_feature.shape
    half = col_table.shape[1]

    out = pl.pallas_call(
        _pe_kernel,
        grid=(B,),
        in_specs=[
            pl.BlockSpec((W, half), lambda b: (0, 0)),
            pl.BlockSpec((H, half), lambda b: (0, 0)),
        ],
        out_specs=pl.BlockSpec((1, C, H * W), lambda b: (b, 0, 0)),
        out_shape=jax.ShapeDtypeStruct((B, C, H * W), jnp.float32),
        scratch_shapes=[pltpu.VMEM((C, H * W), jnp.float32)],
    )(col_table, row_table)
    return out.reshape(B, C, H, W)


# speedup vs baseline: 2.4201x; 1.0054x over previous
import jax
import jax.numpy as jnp
from jax.experimental import pallas as pl
from jax.experimental.pallas import tpu as pltpu


def _pe_kernel(col_ref, row_ref, out_ref, scratch, sem):
    half = col_ref.shape[1]
    W = col_ref.shape[0]
    H = row_ref.shape[0]
    HW = H * W
    B = out_ref.shape[0]

    j = jax.lax.broadcasted_iota(jnp.int32, (W, HW), 1)
    i = jax.lax.broadcasted_iota(jnp.int32, (W, HW), 0)
    s_col = (jax.lax.rem(j, W) == i).astype(jnp.float32)
    s_row = (jax.lax.div(j, W) == i).astype(jnp.float32)
    col_t = col_ref[...].T
    row_t = row_ref[...].T
    scratch[:half] = jax.lax.dot(
        col_t, s_col, precision=jax.lax.Precision.HIGHEST,
        preferred_element_type=jnp.float32)
    scratch[half:] = jax.lax.dot(
        row_t, s_row, precision=jax.lax.Precision.HIGHEST,
        preferred_element_type=jnp.float32)

    copies = [
        pltpu.make_async_copy(scratch, out_ref.at[b], sem) for b in range(B)
    ]
    for c in copies:
        c.start()
    for c in copies:
        c.wait()


def kernel(image_feature, col_table, row_table):
    B, C, H, W = image_feature.shape
    half = col_table.shape[1]

    out = pl.pallas_call(
        _pe_kernel,
        in_specs=[
            pl.BlockSpec(memory_space=pltpu.MemorySpace.VMEM),
            pl.BlockSpec(memory_space=pltpu.MemorySpace.VMEM),
        ],
        out_specs=pl.BlockSpec(memory_space=pltpu.MemorySpace.HBM),
        out_shape=jax.ShapeDtypeStruct((B, C, H * W), jnp.float32),
        scratch_shapes=[
            pltpu.VMEM((C, H * W), jnp.float32),
            pltpu.SemaphoreType.DMA,
        ],
    )(col_table, row_table)
    return out.reshape(B, C, H, W)
